# baseline (speedup 1.0000x reference)
import jax
import jax.numpy as jnp
from jax import lax
from jax.experimental import pallas as pl
from jax.experimental.pallas import tpu as pltpu
from jax.experimental.pallas import tpu_sc as plsc

_NC = 2
_NS = 16
_NW = _NC * _NS

_SEQ = 4096
_D = 2048
_RPW = _SEQ // _NW
_CH = 8
_CW = _CH * _D
_NCHUNK = _RPW // _CH
_BSTRIDE = _SEQ * _D


def _sc_body(x_hbm, pos_hbm, out_hbm, pb, x0, x1, sem):
    wid = lax.axis_index("s") * _NC + lax.axis_index("c")
    base = wid * _RPW * _D

    def chunk(i, _):
        off = base + i * _CW
        cp_p = pltpu.async_copy(pos_hbm.at[pl.ds(off, _CW)], pb, sem)
        cp0 = pltpu.async_copy(x_hbm.at[pl.ds(off, _CW)], x0, sem)
        cp1 = pltpu.async_copy(x_hbm.at[pl.ds(_BSTRIDE + off, _CW)], x1, sem)
        cp_p.wait()
        cp0.wait()
        cp1.wait()

        def body(j, _):
            s = pl.ds(j * 16, 16)
            p = pb[s]
            plsc.addupdate(x0.at[s], p)
            plsc.addupdate(x1.at[s], p)
            return 0

        lax.fori_loop(0, _CW // 16, body, 0, unroll=8)

        o0 = pltpu.async_copy(x0, out_hbm.at[pl.ds(off, _CW)], sem)
        o1 = pltpu.async_copy(x1, out_hbm.at[pl.ds(_BSTRIDE + off, _CW)], sem)
        o0.wait()
        o1.wait()
        return 0

    lax.fori_loop(0, _NCHUNK, chunk, 0)


def kernel(x, pos_table):
    b, s, d = x.shape
    xf = x.reshape(-1)
    pf = pos_table.reshape(-1)
    mesh = plsc.VectorSubcoreMesh(core_axis_name="c", subcore_axis_name="s")
    out = pl.kernel(
        _sc_body,
        out_type=jax.ShapeDtypeStruct((b * s * d,), x.dtype),
        mesh=mesh,
        scratch_types=[
            pltpu.VMEM((_CW,), jnp.float32),
            pltpu.VMEM((_CW,), jnp.float32),
            pltpu.VMEM((_CW,), jnp.float32),
            pltpu.SemaphoreType.DMA,
        ],
    )(xf, pf)
    return out.reshape(b, s, d)


# speedup vs baseline: 1.3048x; 1.3048x over previous
import jax
import jax.numpy as jnp
from jax import lax
from jax.experimental import pallas as pl
from jax.experimental.pallas import tpu as pltpu
from jax.experimental.pallas import tpu_sc as plsc

_NC = 2
_NS = 16
_NW = _NC * _NS

_SEQ = 4096
_D = 2048
_SC_SEQ = 512
_TC_SEQ = _SEQ - _SC_SEQ
_BS = 512

_RPW = _SC_SEQ // _NW
_CH = 8
_CW = _CH * _D
_NCHUNK = _RPW // _CH
_BSTRIDE = _SEQ * _D


def _tc_body(x_ref, pos_ref, out_ref):
    out_ref[...] = x_ref[...] + pos_ref[...][None]


def _sc_body(x_hbm, pos_hbm, out_hbm, pb, x0, x1, sem):
    wid = lax.axis_index("s") * _NC + lax.axis_index("c")
    base = _TC_SEQ * _D + wid * _RPW * _D
    out_base = wid * _RPW * _D

    def chunk(i, _):
        off = base + i * _CW
        ooff = out_base + i * _CW
        cp_p = pltpu.async_copy(pos_hbm.at[pl.ds(off, _CW)], pb, sem)
        cp0 = pltpu.async_copy(x_hbm.at[pl.ds(off, _CW)], x0, sem)
        cp1 = pltpu.async_copy(x_hbm.at[pl.ds(_BSTRIDE + off, _CW)], x1, sem)
        cp_p.wait()
        cp0.wait()
        cp1.wait()

        def body(j, _):
            s = pl.ds(j * 16, 16)
            p = pb[s]
            plsc.addupdate(x0.at[s], p)
            plsc.addupdate(x1.at[s], p)
            return 0

        lax.fori_loop(0, _CW // 16, body, 0, unroll=8)

        o0 = pltpu.async_copy(x0, out_hbm.at[pl.ds(ooff, _CW)], sem)
        o1 = pltpu.async_copy(
            x1, out_hbm.at[pl.ds(_SC_SEQ * _D + ooff, _CW)], sem)
        o0.wait()
        o1.wait()
        return 0

    lax.fori_loop(0, _NCHUNK, chunk, 0)


def kernel(x, pos_table):
    b, s, d = x.shape
    xf = x.reshape(-1)
    pf = pos_table.reshape(-1)

    mesh = plsc.VectorSubcoreMesh(core_axis_name="c", subcore_axis_name="s")
    sc_out = pl.kernel(
        _sc_body,
        out_type=jax.ShapeDtypeStruct((b * _SC_SEQ * d,), x.dtype),
        mesh=mesh,
        scratch_types=[
            pltpu.VMEM((_CW,), jnp.float32),
            pltpu.VMEM((_CW,), jnp.float32),
            pltpu.VMEM((_CW,), jnp.float32),
            pltpu.SemaphoreType.DMA,
        ],
    )(xf, pf)

    tc_out = pl.pallas_call(
        _tc_body,
        grid=(_TC_SEQ // _BS, b),
        in_specs=[
            pl.BlockSpec((1, _BS, d), lambda sblk, bb: (bb, sblk, 0)),
            pl.BlockSpec((_BS, d), lambda sblk, bb: (sblk, 0)),
        ],
        out_specs=pl.BlockSpec((1, _BS, d), lambda sblk, bb: (bb, sblk, 0)),
        out_shape=jax.ShapeDtypeStruct((b, _TC_SEQ, d), x.dtype),
    )(x, pos_table)

    return jnp.concatenate(
        [tc_out, sc_out.reshape(b, _SC_SEQ, d)], axis=1)


# speedup vs baseline: 4.9408x; 3.7866x over previous
import jax
import jax.numpy as jnp
from jax.experimental import pallas as pl


_BS = 512


def _add_body(x_ref, pos_ref, out_ref):
    out_ref[...] = x_ref[...] + pos_ref[...][None]


def kernel(x, pos_table):
    batch, seq, d = x.shape
    return pl.pallas_call(
        _add_body,
        grid=(seq // _BS,),
        in_specs=[
            pl.BlockSpec((batch, _BS, d), lambda s: (0, s, 0)),
            pl.BlockSpec((_BS, d), lambda s: (s, 0)),
        ],
        out_specs=pl.BlockSpec((batch, _BS, d), lambda s: (0, s, 0)),
        out_shape=jax.ShapeDtypeStruct(x.shape, x.dtype),
    )(x, pos_table)
